# Initial kernel scaffold; baseline (speedup 1.0000x reference)
import jax
import jax.numpy as jnp
from jax.experimental import pallas as pl


def kernel(feats, points, W1, g1, b1, W2, g2, b2):
    raise NotImplementedError("write your pallas kernel here")


# speedup vs baseline: 7.7124x; 7.7124x over previous
import functools

import jax
import jax.numpy as jnp
from jax import lax
from jax.experimental import pallas as pl
from jax.experimental.pallas import tpu as pltpu
from jax.experimental.pallas import tpu_sc as plsc

B, CIN, COUT, N = 8, 256, 512, 4096
S = N // 4
K = 16
EPS = 1e-5

TN = 512
NT = N // TN
TS = 256

NC, NS = 2, 16
NW = NC * NS
BAGS = B * S
BPW = BAGS // NW
G = 8
NCHUNK = BPW // G
CHUNK_ROWS = G * K


def _fps_body(pts_ref, cent_ref):
    px = pts_ref[:, 0, :]
    py = pts_ref[:, 1, :]
    pz = pts_ref[:, 2, :]
    ln = lax.broadcasted_iota(jnp.int32, (B, N), 1)
    sn = lax.broadcasted_iota(jnp.int32, (B, S), 1)

    def body(i, carry):
        dists, far, ccx, ccy, ccz = carry
        oh = (ln == far).astype(jnp.float32)
        cx = jnp.sum(px * oh, axis=1, keepdims=True)
        cy = jnp.sum(py * oh, axis=1, keepdims=True)
        cz = jnp.sum(pz * oh, axis=1, keepdims=True)
        sel = sn == i
        ccx = jnp.where(sel, cx, ccx)
        ccy = jnp.where(sel, cy, ccy)
        ccz = jnp.where(sel, cz, ccz)
        d = (px - cx) ** 2 + (py - cy) ** 2 + (pz - cz) ** 2
        dists = jnp.minimum(dists, d)
        m = jnp.max(dists, axis=1, keepdims=True)
        cand = jnp.where(dists == m, ln, N)
        far = jnp.min(cand, axis=1, keepdims=True)
        return dists, far, ccx, ccy, ccz

    dists0 = jnp.full((B, N), 1e10, jnp.float32)
    far0 = jnp.zeros((B, 1), jnp.int32)
    cc0 = jnp.zeros((B, S), jnp.float32)
    _, _, ccx, ccy, ccz = lax.fori_loop(0, S, body, (dists0, far0, cc0, cc0, cc0))
    cent_ref[:, 0, :] = ccx
    cent_ref[:, 1, :] = ccy
    cent_ref[:, 2, :] = ccz


def _fps(points):
    return pl.pallas_call(
        _fps_body,
        out_shape=jax.ShapeDtypeStruct((B, 3, S), jnp.float32),
    )(points)


def _conv1_body(f_ref, w_ref, y_ref, st_ref):
    b = pl.program_id(0)
    t = pl.program_id(1)
    y = lax.dot_general(f_ref[0], w_ref[...], (((0,), (1,)), ((), ())),
                        preferred_element_type=jnp.float32)
    y_ref[0] = y

    @pl.when((b == 0) & (t == 0))
    def _():
        st_ref[...] = jnp.zeros_like(st_ref)

    st_ref[0:1, :] += jnp.sum(y, axis=0, keepdims=True)
    st_ref[1:2, :] += jnp.sum(y * y, axis=0, keepdims=True)


def _conv1(feats, W1):
    return pl.pallas_call(
        _conv1_body,
        grid=(B, NT),
        in_specs=[
            pl.BlockSpec((1, CIN, TN), lambda b, t: (b, 0, t)),
            pl.BlockSpec((COUT, CIN), lambda b, t: (0, 0)),
        ],
        out_specs=[
            pl.BlockSpec((1, TN, COUT), lambda b, t: (b, t, 0)),
            pl.BlockSpec((8, COUT), lambda b, t: (0, 0)),
        ],
        out_shape=[
            jax.ShapeDtypeStruct((B, N, COUT), jnp.float32),
            jax.ShapeDtypeStruct((8, COUT), jnp.float32),
        ],
    )(feats, W1)


def _conv2_body(y1_ref, a_ref, c_ref, w_ref, y_ref, st_ref):
    b = pl.program_id(0)
    t = pl.program_id(1)
    h = jnp.maximum(y1_ref[0] * a_ref[...] + c_ref[...], 0.0)
    y = lax.dot_general(h, w_ref[...], (((1,), (1,)), ((), ())),
                        preferred_element_type=jnp.float32)
    y_ref[0] = y

    @pl.when((b == 0) & (t == 0))
    def _():
        st_ref[...] = jnp.zeros_like(st_ref)

    st_ref[0:1, :] += jnp.sum(y, axis=0, keepdims=True)
    st_ref[1:2, :] += jnp.sum(y * y, axis=0, keepdims=True)


def _conv2(y1, a1, c1, W2):
    return pl.pallas_call(
        _conv2_body,
        grid=(B, NT),
        in_specs=[
            pl.BlockSpec((1, TN, COUT), lambda b, t: (b, t, 0)),
            pl.BlockSpec((1, COUT), lambda b, t: (0, 0)),
            pl.BlockSpec((1, COUT), lambda b, t: (0, 0)),
            pl.BlockSpec((COUT, COUT), lambda b, t: (0, 0)),
        ],
        out_specs=[
            pl.BlockSpec((1, TN, COUT), lambda b, t: (b, t, 0)),
            pl.BlockSpec((8, COUT), lambda b, t: (0, 0)),
        ],
        out_shape=[
            jax.ShapeDtypeStruct((B, N, COUT), jnp.float32),
            jax.ShapeDtypeStruct((8, COUT), jnp.float32),
        ],
    )(y1, a1, c1, W2)


def _knn_body(cent_ref, pts_ref, idx_ref):
    b = pl.program_id(0)
    c = cent_ref[0]
    p = pts_ref[0]
    cn2 = jnp.sum(c * c, axis=0)[:, None]
    pn2 = jnp.sum(p * p, axis=0)[None, :]
    cp = lax.dot_general(c, p, (((0,), (0,)), ((), ())),
                         preferred_element_type=jnp.float32)
    d2 = cn2 + pn2 - 2.0 * cp
    ln = lax.broadcasted_iota(jnp.int32, (TS, N), 1)
    cols = []
    for _ in range(K):
        m = jnp.min(d2, axis=1, keepdims=True)
        cand = jnp.where(d2 == m, ln, N)
        j = jnp.min(cand, axis=1, keepdims=True)
        cols.append(j)
        d2 = jnp.where(ln == j, jnp.inf, d2)
    idx_ref[0] = jnp.concatenate(cols, axis=1) + b * N


def _knn(cent, points):
    return pl.pallas_call(
        _knn_body,
        grid=(B, S // TS),
        in_specs=[
            pl.BlockSpec((1, 3, TS), lambda b, s: (b, 0, s)),
            pl.BlockSpec((1, 3, N), lambda b, s: (b, 0, 0)),
        ],
        out_specs=pl.BlockSpec((1, TS, K), lambda b, s: (b, s, 0)),
        out_shape=jax.ShapeDtypeStruct((B, S, K), jnp.int32),
    )(cent, points)


def _gather_call(table, idxm, ab):
    mesh = plsc.VectorSubcoreMesh(core_axis_name="c", subcore_axis_name="s")

    @functools.partial(
        pl.kernel,
        mesh=mesh,
        out_type=jax.ShapeDtypeStruct((BAGS, COUT), jnp.float32),
        scratch_types=[
            pltpu.VMEM((CHUNK_ROWS,), jnp.int32),
            pltpu.VMEM((CHUNK_ROWS, COUT), jnp.float32),
            pltpu.VMEM((G, COUT), jnp.float32),
            pltpu.VMEM((2, COUT), jnp.float32),
            pltpu.SemaphoreType.DMA,
        ],
    )
    def _sc(table_hbm, idxm_hbm, ab_hbm, out_hbm, idx_v, rows_v, out_v, ab_v, sem):
        wid = lax.axis_index("s") * NC + lax.axis_index("c")
        pltpu.sync_copy(ab_hbm, ab_v)

        def chunk_body(ci, carry):
            gchunk = wid * NCHUNK + ci
            pltpu.sync_copy(idxm_hbm.at[gchunk], idx_v)
            pltpu.async_copy(table_hbm.at[idx_v], rows_v, sem).wait()

            def bag_body(g, c2):
                for j in range(COUT // 16):
                    sl = pl.ds(j * 16, 16)
                    a = ab_v[0, sl]
                    cc = ab_v[1, sl]
                    acc = jnp.zeros((16,), jnp.float32)
                    for r in range(K):
                        x = rows_v[g * K + r, sl]
                        acc = acc + jnp.maximum(x * a + cc, 0.0)
                    out_v[g, sl] = acc * (1.0 / K)
                return c2

            lax.fori_loop(0, G, bag_body, 0)
            pltpu.sync_copy(out_v, out_hbm.at[pl.ds(gchunk * G, G)])
            return carry

        lax.fori_loop(0, NCHUNK, chunk_body, 0)

    return _sc(table, idxm, ab)


def _moments_to_affine(st, g, bb):
    cnt = float(B * N)
    mu = st[0] / cnt
    var = st[1] / cnt - mu * mu
    a = g * lax.rsqrt(var + EPS)
    c = bb - mu * a
    return a, c


def kernel(feats, points, W1, g1, b1, W2, g2, b2):
    cent = _fps(points)
    y1, st1 = _conv1(feats, W1)
    a1, c1 = _moments_to_affine(st1, g1, b1)
    y2, st2 = _conv2(y1, a1[None, :], c1[None, :], W2)
    a2, c2 = _moments_to_affine(st2, g2, b2)
    idx = _knn(cent, points)
    table = y2.reshape(B * N, COUT)
    idxm = idx.reshape(NW * NCHUNK, CHUNK_ROWS)
    ab = jnp.stack([a2, c2])
    outb = _gather_call(table, idxm, ab)
    out = outb.reshape(B, S, COUT).transpose(0, 2, 1)
    return out, cent


# speedup vs baseline: 10.1079x; 1.3106x over previous
import functools

import jax
import jax.numpy as jnp
from jax import lax
from jax.experimental import pallas as pl
from jax.experimental.pallas import tpu as pltpu
from jax.experimental.pallas import tpu_sc as plsc

B, CIN, COUT, N = 8, 256, 512, 4096
S = N // 4
K = 16
EPS = 1e-5

TN = 512
NT = N // TN
TS = 256

NC, NS = 2, 16
NW = NC * NS
BAGS = B * S
BPW = BAGS // NW
G = 4
NCHUNK = BPW // G
CHUNK_ROWS = G * K


def _fps_body(pts_ref, cent_ref):
    px = pts_ref[:, 0, :]
    py = pts_ref[:, 1, :]
    pz = pts_ref[:, 2, :]
    ln = lax.broadcasted_iota(jnp.int32, (B, N), 1)
    lg = lax.broadcasted_iota(jnp.int32, (B, 128), 1)

    def lane_sum(x):
        acc = x[:, 0:128]
        for q in range(1, N // 128):
            acc = acc + x[:, q * 128:(q + 1) * 128]
        return jnp.sum(acc, axis=1, keepdims=True)

    def body(i, carry):
        dists, far = carry
        oh = (ln == far).astype(jnp.float32)
        cx = lane_sum(px * oh)
        cy = lane_sum(py * oh)
        cz = lane_sum(pz * oh)
        g0 = (i // 128) * 128
        sel = (lg + g0) == i
        csl = pl.ds(g0, 128)
        cent_ref[:, 0, csl] = jnp.where(sel, cx, cent_ref[:, 0, csl])
        cent_ref[:, 1, csl] = jnp.where(sel, cy, cent_ref[:, 1, csl])
        cent_ref[:, 2, csl] = jnp.where(sel, cz, cent_ref[:, 2, csl])
        d = (px - cx) ** 2 + (py - cy) ** 2 + (pz - cz) ** 2
        dists = jnp.minimum(dists, d)
        far = jnp.argmax(dists, axis=1).astype(jnp.int32)[:, None]
        return dists, far

    dists0 = jnp.full((B, N), 1e10, jnp.float32)
    far0 = jnp.zeros((B, 1), jnp.int32)
    lax.fori_loop(0, S, body, (dists0, far0))


def _fps(points):
    return pl.pallas_call(
        _fps_body,
        out_shape=jax.ShapeDtypeStruct((B, 3, S), jnp.float32),
    )(points)


def _conv1_body(f_ref, w_ref, y_ref, st_ref):
    b = pl.program_id(0)
    t = pl.program_id(1)
    y = lax.dot_general(f_ref[0], w_ref[...], (((0,), (1,)), ((), ())),
                        preferred_element_type=jnp.float32)
    y_ref[0] = y

    @pl.when((b == 0) & (t == 0))
    def _():
        st_ref[...] = jnp.zeros_like(st_ref)

    st_ref[0:1, :] += jnp.sum(y, axis=0, keepdims=True)
    st_ref[1:2, :] += jnp.sum(y * y, axis=0, keepdims=True)


def _conv1(feats, W1):
    return pl.pallas_call(
        _conv1_body,
        grid=(B, NT),
        in_specs=[
            pl.BlockSpec((1, CIN, TN), lambda b, t: (b, 0, t)),
            pl.BlockSpec((COUT, CIN), lambda b, t: (0, 0)),
        ],
        out_specs=[
            pl.BlockSpec((1, TN, COUT), lambda b, t: (b, t, 0)),
            pl.BlockSpec((8, COUT), lambda b, t: (0, 0)),
        ],
        out_shape=[
            jax.ShapeDtypeStruct((B, N, COUT), jnp.float32),
            jax.ShapeDtypeStruct((8, COUT), jnp.float32),
        ],
    )(feats, W1)


def _conv2_body(y1_ref, a_ref, c_ref, w_ref, y_ref, st_ref):
    b = pl.program_id(0)
    t = pl.program_id(1)
    h = jnp.maximum(y1_ref[0] * a_ref[...] + c_ref[...], 0.0)
    y = lax.dot_general(h, w_ref[...], (((1,), (1,)), ((), ())),
                        preferred_element_type=jnp.float32)
    y_ref[0] = y

    @pl.when((b == 0) & (t == 0))
    def _():
        st_ref[...] = jnp.zeros_like(st_ref)

    st_ref[0:1, :] += jnp.sum(y, axis=0, keepdims=True)
    st_ref[1:2, :] += jnp.sum(y * y, axis=0, keepdims=True)


def _conv2(y1, a1, c1, W2):
    return pl.pallas_call(
        _conv2_body,
        grid=(B, NT),
        in_specs=[
            pl.BlockSpec((1, TN, COUT), lambda b, t: (b, t, 0)),
            pl.BlockSpec((1, COUT), lambda b, t: (0, 0)),
            pl.BlockSpec((1, COUT), lambda b, t: (0, 0)),
            pl.BlockSpec((COUT, COUT), lambda b, t: (0, 0)),
        ],
        out_specs=[
            pl.BlockSpec((1, TN, COUT), lambda b, t: (b, t, 0)),
            pl.BlockSpec((8, COUT), lambda b, t: (0, 0)),
        ],
        out_shape=[
            jax.ShapeDtypeStruct((B, N, COUT), jnp.float32),
            jax.ShapeDtypeStruct((8, COUT), jnp.float32),
        ],
    )(y1, a1, c1, W2)


def _knn_body(cent_ref, pts_ref, idx_ref):
    b = pl.program_id(0)
    c = cent_ref[0]
    p = pts_ref[0]
    cn2 = jnp.sum(c * c, axis=0)[:, None]
    pn2 = jnp.sum(p * p, axis=0)[None, :]
    cp = lax.dot_general(c, p, (((0,), (0,)), ((), ())),
                         preferred_element_type=jnp.float32)
    d2 = cn2 + pn2 - 2.0 * cp
    ln = lax.broadcasted_iota(jnp.int32, (TS, N), 1)
    cols = []
    for _ in range(K):
        j = jnp.argmin(d2, axis=1).astype(jnp.int32)[:, None]
        cols.append(j)
        d2 = jnp.where(ln == j, jnp.inf, d2)
    idx_ref[0] = jnp.concatenate(cols, axis=1) + b * N


def _knn(cent, points):
    sp = cent.shape[2]
    return pl.pallas_call(
        _knn_body,
        grid=(B, sp // TS),
        in_specs=[
            pl.BlockSpec((1, 3, TS), lambda b, s: (b, 0, s)),
            pl.BlockSpec((1, 3, N), lambda b, s: (b, 0, 0)),
        ],
        out_specs=pl.BlockSpec((1, TS, K), lambda b, s: (b, s, 0)),
        out_shape=jax.ShapeDtypeStruct((B, sp, K), jnp.int32),
    )(cent, points)


ROWS = B * N
RPW = ROWS // NW
RCH = 64
NRC = RPW // RCH


def _relu_call(y2f, ab):
    mesh = plsc.VectorSubcoreMesh(core_axis_name="c", subcore_axis_name="s")

    @functools.partial(
        pl.kernel,
        mesh=mesh,
        out_type=jax.ShapeDtypeStruct((ROWS, COUT), jnp.float32),
        scratch_types=[
            pltpu.VMEM((RCH, COUT), jnp.float32),
            pltpu.VMEM((2, COUT), jnp.float32),
        ],
    )
    def _sc(y_hbm, ab_hbm, h_hbm, buf, ab_v):
        wid = lax.axis_index("s") * NC + lax.axis_index("c")
        base = wid * RPW
        pltpu.sync_copy(ab_hbm, ab_v)

        def chunk(ci, carry):
            r0 = base + ci * RCH
            pltpu.sync_copy(y_hbm.at[pl.ds(r0, RCH)], buf)

            def row(r, c2):
                for j in range(COUT // 16):
                    sl = pl.ds(j * 16, 16)
                    buf[r, sl] = jnp.maximum(
                        buf[r, sl] * ab_v[0, sl] + ab_v[1, sl], 0.0)
                return c2

            lax.fori_loop(0, RCH, row, 0)
            pltpu.sync_copy(buf, h_hbm.at[pl.ds(r0, RCH)])
            return carry

        lax.fori_loop(0, NRC, chunk, 0)

    return _sc(y2f, ab)


def _gather_call(table, idx):
    bags = idx.shape[0] * idx.shape[1]
    nch = bags // NW // G
    idxm = idx.reshape(NW, nch, CHUNK_ROWS)
    mesh = plsc.VectorSubcoreMesh(core_axis_name="c", subcore_axis_name="s")

    @functools.partial(
        pl.kernel,
        mesh=mesh,
        out_type=jax.ShapeDtypeStruct((bags, COUT), jnp.float32),
        scratch_types=[
            pltpu.VMEM((nch, CHUNK_ROWS), jnp.int32),
            pltpu.VMEM((CHUNK_ROWS, COUT), jnp.float32),
            pltpu.VMEM((CHUNK_ROWS, COUT), jnp.float32),
            pltpu.VMEM((G, COUT), jnp.float32),
            pltpu.SemaphoreType.DMA,
            pltpu.SemaphoreType.DMA,
        ],
    )
    def _sc(table_hbm, idxm_hbm, out_hbm, idx_v, rows0, rows1, out_v, sem0, sem1):
        wid = lax.axis_index("s") * NC + lax.axis_index("c")
        pltpu.sync_copy(idxm_hbm.at[wid], idx_v)
        pltpu.async_copy(table_hbm.at[idx_v.at[0]], rows0, sem0)

        def compute(ci, rows_v):
            def bag(g, c2):
                for j in range(COUT // 16):
                    sl = pl.ds(j * 16, 16)
                    acc = rows_v[g * K, sl]
                    for r in range(1, K):
                        acc = acc + rows_v[g * K + r, sl]
                    out_v[g, sl] = acc * (1.0 / K)
                return c2

            lax.fori_loop(0, G, bag, 0)
            pltpu.sync_copy(out_v, out_hbm.at[pl.ds((wid * nch + ci) * G, G)])

        def pair(p, carry):
            c0 = p * 2
            pltpu.async_copy(table_hbm.at[idx_v.at[c0 + 1]], rows1, sem1)
            pltpu.make_async_copy(table_hbm.at[idx_v.at[c0]], rows0, sem0).wait()
            compute(c0, rows0)

            @pl.when(c0 + 2 < nch)
            def _():
                pltpu.async_copy(table_hbm.at[idx_v.at[c0 + 2]], rows0, sem0)

            pltpu.make_async_copy(table_hbm.at[idx_v.at[c0 + 1]], rows1, sem1).wait()
            compute(c0 + 1, rows1)
            return carry

        lax.fori_loop(0, nch // 2, pair, 0)

    return _sc(table, idxm)


def _moments_to_affine(st, g, bb):
    cnt = float(B * N)
    mu = st[0] / cnt
    var = st[1] / cnt - mu * mu
    a = g * lax.rsqrt(var + EPS)
    c = bb - mu * a
    return a, c


def kernel(feats, points, W1, g1, b1, W2, g2, b2):
    cent = _fps(points)
    y1, st1 = _conv1(feats, W1)
    a1, c1 = _moments_to_affine(st1, g1, b1)
    y2, st2 = _conv2(y1, a1[None, :], c1[None, :], W2)
    a2, c2 = _moments_to_affine(st2, g2, b2)
    ab = jnp.stack([a2, c2])
    h2 = _relu_call(y2.reshape(B * N, COUT), ab)
    h = S // 2
    idx0 = _knn(cent[:, :, :h], points)
    idx1 = _knn(cent[:, :, h:], points)
    o0 = _gather_call(h2, idx0)
    o1 = _gather_call(h2, idx1)
    outb = jnp.concatenate(
        [o0.reshape(B, h, COUT), o1.reshape(B, h, COUT)], axis=1)
    out = outb.transpose(0, 2, 1)
    return out, cent


# speedup vs baseline: 10.7644x; 1.0649x over previous
import functools

import jax
import jax.numpy as jnp
from jax import lax
from jax.experimental import pallas as pl
from jax.experimental.pallas import tpu as pltpu
from jax.experimental.pallas import tpu_sc as plsc

B, CIN, COUT, N = 8, 256, 512, 4096
S = N // 4
K = 16
EPS = 1e-5

TN = 1024
NT = N // TN
TS = 256

NC, NS = 2, 16
NW = NC * NS
BAGS = B * S
BPW = BAGS // NW
G = 4
NCHUNK = BPW // G
CHUNK_ROWS = G * K


def _fps_body(pts_ref, cent_ref):
    px = pts_ref[:, 0, :]
    py = pts_ref[:, 1, :]
    pz = pts_ref[:, 2, :]
    ln = lax.broadcasted_iota(jnp.int32, (B, N), 1)
    lg = lax.broadcasted_iota(jnp.int32, (B, 128), 1)

    def lane_sum(x):
        parts = [x[:, q * 128:(q + 1) * 128] for q in range(N // 128)]
        while len(parts) > 1:
            half = len(parts) // 2
            parts = [parts[i] + parts[i + half] for i in range(half)]
        return jnp.sum(parts[0], axis=1, keepdims=True)

    def body(i, carry):
        dists, far = carry
        oh = (ln == far).astype(jnp.float32)
        cx = lane_sum(px * oh)
        cy = lane_sum(py * oh)
        cz = lane_sum(pz * oh)
        g0 = (i // 128) * 128
        sel = (lg + g0) == i
        csl = pl.ds(g0, 128)
        cent_ref[:, 0, csl] = jnp.where(sel, cx, cent_ref[:, 0, csl])
        cent_ref[:, 1, csl] = jnp.where(sel, cy, cent_ref[:, 1, csl])
        cent_ref[:, 2, csl] = jnp.where(sel, cz, cent_ref[:, 2, csl])
        d = (px - cx) ** 2 + (py - cy) ** 2 + (pz - cz) ** 2
        dists = jnp.minimum(dists, d)
        far = jnp.argmax(dists, axis=1).astype(jnp.int32)[:, None]
        return dists, far

    dists0 = jnp.full((B, N), 1e10, jnp.float32)
    far0 = jnp.zeros((B, 1), jnp.int32)
    lax.fori_loop(0, S, body, (dists0, far0))


def _fps(points):
    return pl.pallas_call(
        _fps_body,
        out_shape=jax.ShapeDtypeStruct((B, 3, S), jnp.float32),
    )(points)


def _conv1_body(f_ref, w_ref, y_ref, st_ref):
    b = pl.program_id(0)
    t = pl.program_id(1)
    y = lax.dot_general(f_ref[0], w_ref[...], (((0,), (1,)), ((), ())),
                        preferred_element_type=jnp.float32)
    y_ref[0] = y

    @pl.when((b == 0) & (t == 0))
    def _():
        st_ref[...] = jnp.zeros_like(st_ref)

    st_ref[0:1, :] += jnp.sum(y, axis=0, keepdims=True)
    st_ref[1:2, :] += jnp.sum(y * y, axis=0, keepdims=True)


def _conv1(feats, W1):
    return pl.pallas_call(
        _conv1_body,
        grid=(B, NT),
        in_specs=[
            pl.BlockSpec((1, CIN, TN), lambda b, t: (b, 0, t)),
            pl.BlockSpec((COUT, CIN), lambda b, t: (0, 0)),
        ],
        out_specs=[
            pl.BlockSpec((1, TN, COUT), lambda b, t: (b, t, 0)),
            pl.BlockSpec((8, COUT), lambda b, t: (0, 0)),
        ],
        out_shape=[
            jax.ShapeDtypeStruct((B, N, COUT), jnp.float32),
            jax.ShapeDtypeStruct((8, COUT), jnp.float32),
        ],
    )(feats, W1)


def _conv2_body(y1_ref, a_ref, c_ref, w_ref, y_ref, st_ref):
    b = pl.program_id(0)
    t = pl.program_id(1)
    h = jnp.maximum(y1_ref[0] * a_ref[...] + c_ref[...], 0.0)
    y = lax.dot_general(h, w_ref[...], (((1,), (1,)), ((), ())),
                        preferred_element_type=jnp.float32)
    y_ref[0] = y

    @pl.when((b == 0) & (t == 0))
    def _():
        st_ref[...] = jnp.zeros_like(st_ref)

    st_ref[0:1, :] += jnp.sum(y, axis=0, keepdims=True)
    st_ref[1:2, :] += jnp.sum(y * y, axis=0, keepdims=True)


def _conv2(y1, a1, c1, W2):
    return pl.pallas_call(
        _conv2_body,
        grid=(B, NT),
        in_specs=[
            pl.BlockSpec((1, TN, COUT), lambda b, t: (b, t, 0)),
            pl.BlockSpec((1, COUT), lambda b, t: (0, 0)),
            pl.BlockSpec((1, COUT), lambda b, t: (0, 0)),
            pl.BlockSpec((COUT, COUT), lambda b, t: (0, 0)),
        ],
        out_specs=[
            pl.BlockSpec((1, TN, COUT), lambda b, t: (b, t, 0)),
            pl.BlockSpec((8, COUT), lambda b, t: (0, 0)),
        ],
        out_shape=[
            jax.ShapeDtypeStruct((B, N, COUT), jnp.float32),
            jax.ShapeDtypeStruct((8, COUT), jnp.float32),
        ],
    )(y1, a1, c1, W2)


def _knn_body(cent_ref, pts_ref, idx_ref):
    b = pl.program_id(0)
    c = cent_ref[0]
    p = pts_ref[0]
    cn2 = jnp.sum(c * c, axis=0)[:, None]
    pn2 = jnp.sum(p * p, axis=0)[None, :]
    cp = lax.dot_general(c, p, (((0,), (0,)), ((), ())),
                         preferred_element_type=jnp.float32)
    d2 = cn2 + pn2 - 2.0 * cp
    ln = lax.broadcasted_iota(jnp.int32, (TS, N), 1)
    cols = []
    for _ in range(K):
        j = jnp.argmin(d2, axis=1).astype(jnp.int32)[:, None]
        cols.append(j)
        d2 = jnp.where(ln == j, jnp.inf, d2)
    idx_ref[0] = jnp.concatenate(cols, axis=1) + b * N


def _knn(cent, points):
    sp = cent.shape[2]
    return pl.pallas_call(
        _knn_body,
        grid=(B, sp // TS),
        in_specs=[
            pl.BlockSpec((1, 3, TS), lambda b, s: (b, 0, s)),
            pl.BlockSpec((1, 3, N), lambda b, s: (b, 0, 0)),
        ],
        out_specs=pl.BlockSpec((1, TS, K), lambda b, s: (b, s, 0)),
        out_shape=jax.ShapeDtypeStruct((B, sp, K), jnp.int32),
    )(cent, points)


ROWS = B * N
RPW = ROWS // NW
RCH = 64
NRC = RPW // RCH


def _relu_call(y2f, ab):
    mesh = plsc.VectorSubcoreMesh(core_axis_name="c", subcore_axis_name="s")

    @functools.partial(
        pl.kernel,
        mesh=mesh,
        out_type=jax.ShapeDtypeStruct((ROWS, COUT), jnp.float32),
        scratch_types=[
            pltpu.VMEM((RCH, COUT), jnp.float32),
            pltpu.VMEM((2, COUT), jnp.float32),
        ],
    )
    def _sc(y_hbm, ab_hbm, h_hbm, buf, ab_v):
        wid = lax.axis_index("s") * NC + lax.axis_index("c")
        base = wid * RPW
        pltpu.sync_copy(ab_hbm, ab_v)

        def chunk(ci, carry):
            r0 = base + ci * RCH
            pltpu.sync_copy(y_hbm.at[pl.ds(r0, RCH)], buf)

            def row(r, c2):
                for j in range(COUT // 16):
                    sl = pl.ds(j * 16, 16)
                    buf[r, sl] = jnp.maximum(
                        buf[r, sl] * ab_v[0, sl] + ab_v[1, sl], 0.0)
                return c2

            lax.fori_loop(0, RCH, row, 0)
            pltpu.sync_copy(buf, h_hbm.at[pl.ds(r0, RCH)])
            return carry

        lax.fori_loop(0, NRC, chunk, 0)

    return _sc(y2f, ab)


def _gather_call(table, idx):
    bags = idx.shape[0] * idx.shape[1]
    nch = bags // NW // G
    idxm = idx.reshape(NW, nch, CHUNK_ROWS)
    mesh = plsc.VectorSubcoreMesh(core_axis_name="c", subcore_axis_name="s")

    @functools.partial(
        pl.kernel,
        mesh=mesh,
        out_type=jax.ShapeDtypeStruct((bags, COUT), jnp.float32),
        scratch_types=[
            pltpu.VMEM((nch, CHUNK_ROWS), jnp.int32),
            pltpu.VMEM((CHUNK_ROWS, COUT), jnp.float32),
            pltpu.VMEM((CHUNK_ROWS, COUT), jnp.float32),
            pltpu.VMEM((G, COUT), jnp.float32),
            pltpu.SemaphoreType.DMA,
            pltpu.SemaphoreType.DMA,
        ],
    )
    def _sc(table_hbm, idxm_hbm, out_hbm, idx_v, rows0, rows1, out_v, sem0, sem1):
        wid = lax.axis_index("s") * NC + lax.axis_index("c")
        pltpu.sync_copy(idxm_hbm.at[wid], idx_v)
        pltpu.async_copy(table_hbm.at[idx_v.at[0]], rows0, sem0)

        def compute(ci, rows_v):
            def bag(g, c2):
                for j in range(COUT // 16):
                    sl = pl.ds(j * 16, 16)
                    acc = rows_v[g * K, sl]
                    for r in range(1, K):
                        acc = acc + rows_v[g * K + r, sl]
                    out_v[g, sl] = acc * (1.0 / K)
                return c2

            lax.fori_loop(0, G, bag, 0)
            pltpu.sync_copy(out_v, out_hbm.at[pl.ds((wid * nch + ci) * G, G)])

        def pair(p, carry):
            c0 = p * 2
            pltpu.async_copy(table_hbm.at[idx_v.at[c0 + 1]], rows1, sem1)
            pltpu.make_async_copy(table_hbm.at[idx_v.at[c0]], rows0, sem0).wait()
            compute(c0, rows0)

            @pl.when(c0 + 2 < nch)
            def _():
                pltpu.async_copy(table_hbm.at[idx_v.at[c0 + 2]], rows0, sem0)

            pltpu.make_async_copy(table_hbm.at[idx_v.at[c0 + 1]], rows1, sem1).wait()
            compute(c0 + 1, rows1)
            return carry

        lax.fori_loop(0, nch // 2, pair, 0)

    return _sc(table, idxm)


def _moments_to_affine(st, g, bb):
    cnt = float(B * N)
    mu = st[0] / cnt
    var = st[1] / cnt - mu * mu
    a = g * lax.rsqrt(var + EPS)
    c = bb - mu * a
    return a, c


def kernel(feats, points, W1, g1, b1, W2, g2, b2):
    cent = _fps(points)
    y1, st1 = _conv1(feats, W1)
    a1, c1 = _moments_to_affine(st1, g1, b1)
    y2, st2 = _conv2(y1, a1[None, :], c1[None, :], W2)
    a2, c2 = _moments_to_affine(st2, g2, b2)
    ab = jnp.stack([a2, c2])
    h2 = _relu_call(y2.reshape(B * N, COUT), ab)
    h = S // 4
    idxs = [_knn(cent[:, :, i * h:(i + 1) * h], points) for i in range(4)]
    outs = [_gather_call(h2, ix) for ix in idxs]
    outb = jnp.concatenate([o.reshape(B, h, COUT) for o in outs], axis=1)
    out = outb.transpose(0, 2, 1)
    return out, cent


# speedup vs baseline: 10.8178x; 1.0050x over previous
import functools

import jax
import jax.numpy as jnp
from jax import lax
from jax.experimental import pallas as pl
from jax.experimental.pallas import tpu as pltpu
from jax.experimental.pallas import tpu_sc as plsc

B, CIN, COUT, N = 8, 256, 512, 4096
S = N // 4
K = 16
EPS = 1e-5

TN = 1024
NT = N // TN
TS = 256

NC, NS = 2, 16
NW = NC * NS
BAGS = B * S
BPW = BAGS // NW
G = 4
NCHUNK = BPW // G
CHUNK_ROWS = G * K


def _fps_body(pts_ref, cent_ref):
    px = pts_ref[:, 0, :]
    py = pts_ref[:, 1, :]
    pz = pts_ref[:, 2, :]
    ln = lax.broadcasted_iota(jnp.int32, (B, N), 1)
    lg = lax.broadcasted_iota(jnp.int32, (B, 128), 1)

    def body(i, carry):
        dists, far = carry
        oh = (ln == far).astype(jnp.float32)
        cx = jnp.sum(px * oh, axis=1, keepdims=True)
        cy = jnp.sum(py * oh, axis=1, keepdims=True)
        cz = jnp.sum(pz * oh, axis=1, keepdims=True)
        g0 = (i // 128) * 128
        sel = (lg + g0) == i
        csl = pl.ds(g0, 128)
        cent_ref[:, 0, csl] = jnp.where(sel, cx, cent_ref[:, 0, csl])
        cent_ref[:, 1, csl] = jnp.where(sel, cy, cent_ref[:, 1, csl])
        cent_ref[:, 2, csl] = jnp.where(sel, cz, cent_ref[:, 2, csl])
        d = (px - cx) ** 2 + (py - cy) ** 2 + (pz - cz) ** 2
        dists = jnp.minimum(dists, d)
        m = jnp.max(dists, axis=1, keepdims=True)
        cand = jnp.where(dists == m, ln, N)
        far = jnp.min(cand, axis=1, keepdims=True)
        return dists, far

    dists0 = jnp.full((B, N), 1e10, jnp.float32)
    far0 = jnp.zeros((B, 1), jnp.int32)
    lax.fori_loop(0, S, body, (dists0, far0))


def _fps(points):
    return pl.pallas_call(
        _fps_body,
        out_shape=jax.ShapeDtypeStruct((B, 3, S), jnp.float32),
    )(points)


def _conv1_body(f_ref, w_ref, y_ref, st_ref):
    b = pl.program_id(0)
    t = pl.program_id(1)
    y = lax.dot_general(f_ref[0], w_ref[...], (((0,), (1,)), ((), ())),
                        preferred_element_type=jnp.float32)
    y_ref[0] = y

    @pl.when((b == 0) & (t == 0))
    def _():
        st_ref[...] = jnp.zeros_like(st_ref)

    st_ref[0:1, :] += jnp.sum(y, axis=0, keepdims=True)
    st_ref[1:2, :] += jnp.sum(y * y, axis=0, keepdims=True)


def _conv1(feats, W1):
    return pl.pallas_call(
        _conv1_body,
        grid=(B, NT),
        in_specs=[
            pl.BlockSpec((1, CIN, TN), lambda b, t: (b, 0, t)),
            pl.BlockSpec((COUT, CIN), lambda b, t: (0, 0)),
        ],
        out_specs=[
            pl.BlockSpec((1, TN, COUT), lambda b, t: (b, t, 0)),
            pl.BlockSpec((8, COUT), lambda b, t: (0, 0)),
        ],
        out_shape=[
            jax.ShapeDtypeStruct((B, N, COUT), jnp.float32),
            jax.ShapeDtypeStruct((8, COUT), jnp.float32),
        ],
    )(feats, W1)


def _conv2_body(y1_ref, a_ref, c_ref, w_ref, y_ref, st_ref):
    b = pl.program_id(0)
    t = pl.program_id(1)
    h = jnp.maximum(y1_ref[0] * a_ref[...] + c_ref[...], 0.0)
    y = lax.dot_general(h, w_ref[...], (((1,), (1,)), ((), ())),
                        preferred_element_type=jnp.float32)
    y_ref[0] = y

    @pl.when((b == 0) & (t == 0))
    def _():
        st_ref[...] = jnp.zeros_like(st_ref)

    st_ref[0:1, :] += jnp.sum(y, axis=0, keepdims=True)
    st_ref[1:2, :] += jnp.sum(y * y, axis=0, keepdims=True)


def _conv2(y1, a1, c1, W2):
    return pl.pallas_call(
        _conv2_body,
        grid=(B, NT),
        in_specs=[
            pl.BlockSpec((1, TN, COUT), lambda b, t: (b, t, 0)),
            pl.BlockSpec((1, COUT), lambda b, t: (0, 0)),
            pl.BlockSpec((1, COUT), lambda b, t: (0, 0)),
            pl.BlockSpec((COUT, COUT), lambda b, t: (0, 0)),
        ],
        out_specs=[
            pl.BlockSpec((1, TN, COUT), lambda b, t: (b, t, 0)),
            pl.BlockSpec((8, COUT), lambda b, t: (0, 0)),
        ],
        out_shape=[
            jax.ShapeDtypeStruct((B, N, COUT), jnp.float32),
            jax.ShapeDtypeStruct((8, COUT), jnp.float32),
        ],
    )(y1, a1, c1, W2)


def _knn_body(cent_ref, pts_ref, idx_ref):
    b = pl.program_id(0)
    c = cent_ref[0]
    p = pts_ref[0]
    cn2 = jnp.sum(c * c, axis=0)[:, None]
    pn2 = jnp.sum(p * p, axis=0)[None, :]
    cp = lax.dot_general(c, p, (((0,), (0,)), ((), ())),
                         preferred_element_type=jnp.float32)
    d2 = cn2 + pn2 - 2.0 * cp
    ln = lax.broadcasted_iota(jnp.int32, (TS, N), 1)
    cols = []
    for _ in range(K):
        j = jnp.argmin(d2, axis=1).astype(jnp.int32)[:, None]
        cols.append(j)
        d2 = jnp.where(ln == j, jnp.inf, d2)
    idx_ref[0] = jnp.concatenate(cols, axis=1) + b * N


def _knn(cent, points):
    sp = cent.shape[2]
    return pl.pallas_call(
        _knn_body,
        grid=(B, sp // TS),
        in_specs=[
            pl.BlockSpec((1, 3, TS), lambda b, s: (b, 0, s)),
            pl.BlockSpec((1, 3, N), lambda b, s: (b, 0, 0)),
        ],
        out_specs=pl.BlockSpec((1, TS, K), lambda b, s: (b, s, 0)),
        out_shape=jax.ShapeDtypeStruct((B, sp, K), jnp.int32),
    )(cent, points)


ROWS = B * N
RPW = ROWS // NW
RCH = 64
NRC = RPW // RCH


def _relu_call(y2f, ab):
    mesh = plsc.VectorSubcoreMesh(core_axis_name="c", subcore_axis_name="s")

    @functools.partial(
        pl.kernel,
        mesh=mesh,
        out_type=jax.ShapeDtypeStruct((ROWS, COUT), jnp.float32),
        scratch_types=[
            pltpu.VMEM((RCH, COUT), jnp.float32),
            pltpu.VMEM((2, COUT), jnp.float32),
        ],
    )
    def _sc(y_hbm, ab_hbm, h_hbm, buf, ab_v):
        wid = lax.axis_index("s") * NC + lax.axis_index("c")
        base = wid * RPW
        pltpu.sync_copy(ab_hbm, ab_v)

        def chunk(ci, carry):
            r0 = base + ci * RCH
            pltpu.sync_copy(y_hbm.at[pl.ds(r0, RCH)], buf)

            def row(r, c2):
                for j in range(COUT // 16):
                    sl = pl.ds(j * 16, 16)
                    buf[r, sl] = jnp.maximum(
                        buf[r, sl] * ab_v[0, sl] + ab_v[1, sl], 0.0)
                return c2

            lax.fori_loop(0, RCH, row, 0)
            pltpu.sync_copy(buf, h_hbm.at[pl.ds(r0, RCH)])
            return carry

        lax.fori_loop(0, NRC, chunk, 0)

    return _sc(y2f, ab)


def _gather_call(table, idx):
    bags = idx.shape[0] * idx.shape[1]
    nch = bags // NW // G
    idxm = idx.reshape(NW, nch, CHUNK_ROWS)
    mesh = plsc.VectorSubcoreMesh(core_axis_name="c", subcore_axis_name="s")

    @functools.partial(
        pl.kernel,
        mesh=mesh,
        out_type=jax.ShapeDtypeStruct((bags, COUT), jnp.float32),
        scratch_types=[
            pltpu.VMEM((nch, CHUNK_ROWS), jnp.int32),
            pltpu.VMEM((CHUNK_ROWS, COUT), jnp.float32),
            pltpu.VMEM((CHUNK_ROWS, COUT), jnp.float32),
            pltpu.VMEM((G, COUT), jnp.float32),
            pltpu.SemaphoreType.DMA,
            pltpu.SemaphoreType.DMA,
        ],
    )
    def _sc(table_hbm, idxm_hbm, out_hbm, idx_v, rows0, rows1, out_v, sem0, sem1):
        wid = lax.axis_index("s") * NC + lax.axis_index("c")
        pltpu.sync_copy(idxm_hbm.at[wid], idx_v)
        pltpu.async_copy(table_hbm.at[idx_v.at[0]], rows0, sem0)

        def compute(ci, rows_v):
            def bag(g, c2):
                for j in range(COUT // 16):
                    sl = pl.ds(j * 16, 16)
                    acc = rows_v[g * K, sl]
                    for r in range(1, K):
                        acc = acc + rows_v[g * K + r, sl]
                    out_v[g, sl] = acc * (1.0 / K)
                return c2

            lax.fori_loop(0, G, bag, 0)
            pltpu.sync_copy(out_v, out_hbm.at[pl.ds((wid * nch + ci) * G, G)])

        def pair(p, carry):
            c0 = p * 2
            pltpu.async_copy(table_hbm.at[idx_v.at[c0 + 1]], rows1, sem1)
            pltpu.make_async_copy(table_hbm.at[idx_v.at[c0]], rows0, sem0).wait()
            compute(c0, rows0)

            @pl.when(c0 + 2 < nch)
            def _():
                pltpu.async_copy(table_hbm.at[idx_v.at[c0 + 2]], rows0, sem0)

            pltpu.make_async_copy(table_hbm.at[idx_v.at[c0 + 1]], rows1, sem1).wait()
            compute(c0 + 1, rows1)
            return carry

        lax.fori_loop(0, nch // 2, pair, 0)

    return _sc(table, idxm)


def _moments_to_affine(st, g, bb):
    cnt = float(B * N)
    mu = st[0] / cnt
    var = st[1] / cnt - mu * mu
    a = g * lax.rsqrt(var + EPS)
    c = bb - mu * a
    return a, c


def kernel(feats, points, W1, g1, b1, W2, g2, b2):
    cent = _fps(points)
    y1, st1 = _conv1(feats, W1)
    a1, c1 = _moments_to_affine(st1, g1, b1)
    y2, st2 = _conv2(y1, a1[None, :], c1[None, :], W2)
    a2, c2 = _moments_to_affine(st2, g2, b2)
    ab = jnp.stack([a2, c2])
    h2 = _relu_call(y2.reshape(B * N, COUT), ab)
    h = S // 4
    idxs = [_knn(cent[:, :, i * h:(i + 1) * h], points) for i in range(4)]
    outs = [_gather_call(h2, ix) for ix in idxs]
    outb = jnp.concatenate([o.reshape(B, h, COUT) for o in outs], axis=1)
    out = outb.transpose(0, 2, 1)
    return out, cent
